# baseline (speedup 1.0000x reference)
import functools

import jax
import jax.numpy as jnp
from jax import lax
from jax.experimental import pallas as pl
from jax.experimental.pallas import tpu as pltpu
from jax.experimental.pallas import tpu_sc as plsc

D = 64
NC = 2
NS = 16
NW = NC * NS
G = 128
GPC = 2
CHUNK = G * GPC
NBUF = 4


def _embed_call(B, V):
    b_per_w = B // NW
    n_gather = b_per_w // G
    n_chunks = b_per_w // CHUNK
    assert n_chunks % NBUF == 0
    mesh = plsc.VectorSubcoreMesh(
        core_axis_name="c", subcore_axis_name="s",
        num_cores=NC, num_subcores=NS)

    @functools.partial(
        pl.kernel,
        mesh=mesh,
        compiler_params=pltpu.CompilerParams(use_tc_tiling_on_sc=False),
        out_type=jax.ShapeDtypeStruct((B, D), jnp.float32),
        scratch_types=[
            pltpu.VMEM((n_gather, G), jnp.int32),
            pltpu.VMEM((NBUF, CHUNK, D), jnp.float32),
            [pltpu.SemaphoreType.DMA] * NBUF,
            [pltpu.SemaphoreType.DMA] * NBUF,
        ],
    )
    def k(idx_hbm, table_hbm, out_hbm, idx_v, bufs, sems_g, sems_s):
        wid = lax.axis_index("s") * NC + lax.axis_index("c")
        base = wid * b_per_w
        pltpu.sync_copy(idx_hbm.at[wid], idx_v)

        def issue_gathers(c, p):
            for j in range(GPC):
                pltpu.async_copy(
                    table_hbm.at[idx_v.at[c * GPC + j]],
                    bufs.at[p, pl.ds(j * G, G)],
                    sems_g[p])

        def wait_gathers(p):
            pltpu.make_async_copy(
                table_hbm.at[pl.ds(0, CHUNK)], bufs.at[p], sems_g[p]).wait()

        def issue_store(c, p):
            pltpu.async_copy(
                bufs.at[p], out_hbm.at[pl.ds(base + c * CHUNK, CHUNK)],
                sems_s[p])

        def wait_store(p):
            pltpu.make_async_copy(
                bufs.at[p], out_hbm.at[pl.ds(0, CHUNK)], sems_s[p]).wait()

        for c in range(NBUF - 1):
            issue_gathers(c, c)

        def body(i, _):
            c0 = i * NBUF
            for p in range(NBUF):
                c = c0 + p
                q = (p + NBUF - 1) % NBUF

                @pl.when(c > 0)
                def _():
                    wait_store(q)

                @pl.when(c + NBUF - 1 < n_chunks)
                def _():
                    issue_gathers(c + NBUF - 1, q)

                wait_gathers(p)
                issue_store(c, p)
            return 0

        lax.fori_loop(0, n_chunks // NBUF, body, 0)

        wait_store((n_chunks - 1) % NBUF)

    return k


def kernel(x, weight):
    orig_shape = x.shape
    idx = x.reshape(-1).astype(jnp.int32)
    B = idx.shape[0]
    idx3 = idx.reshape(NW, (B // NW) // G, G)
    out = _embed_call(B, weight.shape[0])(idx3, weight)
    return out.reshape(*orig_shape, D)
